# baseline (speedup 1.0000x reference)
import functools

import jax
import jax.numpy as jnp
from jax import lax
from jax.experimental import pallas as pl
from jax.experimental.pallas import tpu as pltpu
from jax.experimental.pallas import tpu_sc as plsc

_N = 10000
_E = 320000
_HID = 256
_ROWS_BLK = 2000

_NSC = 2
_NT = 16
_NR = 20480
_RT = _NR // _NT
_EP = _E // _NT
_K = 128
_NCHT = 158
_EPAD = _NT * _NCHT * _K - _E
_SB = 80


def _sc_fused_body(xs, pk1, pk2, s_out, t_out, acc, hist, pkA, pkB, rowsA,
                   rowsB, sb, histv, onesv, semA, semB, semSA, semSB, semHA,
                   semHB):
    c = lax.axis_index("c")
    s = lax.axis_index("s")
    r0 = s * _RT
    cb = (c * _NT + s) * _NCHT
    z16 = jnp.zeros((16,), jnp.float32)
    one16 = jnp.ones((16,), jnp.float32)
    n_strips = _RT // _SB
    lane_splats = [jnp.full((16, 1), r, jnp.int32) for r in range(16)]
    _gd = lax.GatherDimensionNumbers(
        offset_dims=(), collapsed_slice_dims=(0,), start_index_map=(0,))

    def _zero_acc_hist():
        def _zstrip(st, carry):
            pltpu.sync_copy(rowsA.at[pl.ds(0, _SB)],
                            acc.at[pl.ds(r0 + st * _SB, _SB)])
            return carry
        lax.fori_loop(0, n_strips, _zstrip, 0)
        pltpu.sync_copy(histv, hist.at[pl.ds(r0, _RT)])

    def _edge_loop(table, pk):
        pltpu.sync_copy(pk.at[cb], pkA)
        pltpu.async_copy(table.at[pkA.at[0]], rowsA, semA)
        pltpu.sync_copy(pk.at[cb + 1], pkB)
        pltpu.async_copy(table.at[pkB.at[0]], rowsB, semB)

        def _pair(p, carry):
            a = 2 * p
            pltpu.make_async_copy(table.at[pl.ds(0, _K)], rowsA, semA).wait()
            sA = pltpu.async_copy(rowsA, acc.at[pkA.at[1]], semSA, add=True)
            hA = pltpu.async_copy(onesv, hist.at[pkA.at[1]], semHA, add=True)

            pltpu.make_async_copy(table.at[pl.ds(0, _K)], rowsB, semB).wait()
            sB = pltpu.async_copy(rowsB, acc.at[pkB.at[1]], semSB, add=True)
            hB = pltpu.async_copy(onesv, hist.at[pkB.at[1]], semHB, add=True)

            sA.wait()
            hA.wait()

            @pl.when(a + 2 < _NCHT)
            def _():
                pltpu.sync_copy(pk.at[cb + a + 2], pkA)
                pltpu.async_copy(table.at[pkA.at[0]], rowsA, semA)

            sB.wait()
            hB.wait()

            @pl.when(a + 3 < _NCHT)
            def _():
                pltpu.sync_copy(pk.at[cb + a + 3], pkB)
                pltpu.async_copy(table.at[pkB.at[0]], rowsB, semB)
            return carry
        lax.fori_loop(0, _NCHT // 2, _pair, 0)

    def _finalize(dst, rezero):
        if rezero:
            def _rz(i, carry):
                for j in range(4):
                    rowsA[i, pl.ds(j * 16, 16)] = z16
                return carry
            lax.fori_loop(0, _SB, _rz, 0)
        pltpu.sync_copy(hist.at[pl.ds(r0, _RT)], histv)

        def _inv(g, carry):
            hv = histv[pl.ds(g * 16, 16)]
            histv[pl.ds(g * 16, 16)] = jnp.where(hv > 0.0, 1.0 / hv, 0.0)
            return carry
        lax.fori_loop(0, _RT // 16, _inv, 0)

        def _fstrip(st, carry):
            pltpu.sync_copy(acc.at[pl.ds(r0 + st * _SB, _SB)], sb)

            def _grp(g, carry2):
                inv16 = histv[pl.ds(st * _SB + g * 16, 16)]
                for r in range(16):
                    splat = lax.gather(
                        inv16, lane_splats[r], _gd, slice_sizes=(1,),
                        mode=lax.GatherScatterMode.PROMISE_IN_BOUNDS)
                    row = g * 16 + r
                    for j in range(4):
                        sb[row, pl.ds(j * 16, 16)] = sb[row, pl.ds(j * 16, 16)] * splat
                return carry2
            lax.fori_loop(0, _SB // 16, _grp, 0)
            pltpu.sync_copy(sb, dst.at[pl.ds(c * _NR + r0 + st * _SB, _SB)])
            if rezero:
                pltpu.sync_copy(rowsA.at[pl.ds(0, _SB)],
                                acc.at[pl.ds(r0 + st * _SB, _SB)])
            return carry
        lax.fori_loop(0, n_strips, _fstrip, 0)
        if rezero:
            def _zh(g, carry):
                histv[pl.ds(g * 16, 16)] = z16
                return carry
            lax.fori_loop(0, _RT // 16, _zh, 0)
            pltpu.sync_copy(histv, hist.at[pl.ds(r0, _RT)])

    def _zrow(i, carry):
        for j in range(4):
            rowsA[i, pl.ds(j * 16, 16)] = z16
        return carry
    lax.fori_loop(0, _K, _zrow, 0)

    def _zhist(i, carry):
        histv[pl.ds(i * 16, 16)] = z16
        return carry
    lax.fori_loop(0, _RT // 16, _zhist, 0)
    for j in range(_K // 16):
        onesv[pl.ds(j * 16, 16)] = one16
    _zero_acc_hist()
    plsc.subcore_barrier()

    _edge_loop(xs, pk1)
    plsc.subcore_barrier()
    _finalize(t_out, rezero=True)
    plsc.subcore_barrier()

    _edge_loop(t_out, pk2)
    plsc.subcore_barrier()
    _finalize(s_out, rezero=False)


def _sc_fused(xs, pk1, pk2):
    mesh = plsc.VectorSubcoreMesh(core_axis_name="c", subcore_axis_name="s")
    f = pl.kernel(
        _sc_fused_body,
        mesh=mesh,
        out_type=[
            jax.ShapeDtypeStruct((_NSC * _NR, 64), jnp.float32),
            jax.ShapeDtypeStruct((_NSC * _NR, 64), jnp.float32),
        ],
        scratch_types=[
            pltpu.VMEM_SHARED((_NR, 64), jnp.float32),
            pltpu.VMEM_SHARED((_NR,), jnp.float32),
            pltpu.VMEM((2, _K), jnp.int32),
            pltpu.VMEM((2, _K), jnp.int32),
            pltpu.VMEM((_K, 64), jnp.float32),
            pltpu.VMEM((_K, 64), jnp.float32),
            pltpu.VMEM((_SB, 64), jnp.float32),
            pltpu.VMEM((_RT,), jnp.float32),
            pltpu.VMEM((_K,), jnp.float32),
            pltpu.SemaphoreType.DMA,
            pltpu.SemaphoreType.DMA,
            pltpu.SemaphoreType.DMA,
            pltpu.SemaphoreType.DMA,
            pltpu.SemaphoreType.DMA,
            pltpu.SemaphoreType.DMA,
        ],
        compiler_params=pltpu.CompilerParams(use_tc_tiling_on_sc=False),
    )
    return f(xs, pk1, pk2)


def _dense_body(s_ref, W2_ref, b2_ref, mixW_ref, mixb_ref,
                Wih_ref, bih_ref, bhh_ref, roW_ref, rob_ref, h_ref, o_ref):
    s = s_ref[...]
    u = jnp.dot(s, W2_ref[...], preferred_element_type=jnp.float32) + b2_ref[...]
    h = jnp.maximum(
        jnp.dot(u, mixW_ref[...], preferred_element_type=jnp.float32) + mixb_ref[...],
        0.0)
    gi = jnp.dot(h, Wih_ref[...], preferred_element_type=jnp.float32) + bih_ref[...]
    bhh = bhh_ref[...]
    r = jax.nn.sigmoid(gi[:, 0:_HID] + bhh[:, 0:_HID])
    z = jax.nn.sigmoid(gi[:, _HID:2 * _HID] + bhh[:, _HID:2 * _HID])
    n = jnp.tanh(gi[:, 2 * _HID:] + r * bhh[:, 2 * _HID:])
    hn = (1.0 - z) * n
    h_ref[...] = hn
    o_ref[...] = jnp.dot(hn, roW_ref[...], preferred_element_type=jnp.float32) + rob_ref[...]


def _dense_stage(s_cat, W2, b2, mix_W, mix_b, Wih, bih, bhh, ro_W, ro_b):
    grid = (_N // _ROWS_BLK,)
    full = lambda shape: pl.BlockSpec(shape, lambda i: (0, 0))
    return pl.pallas_call(
        _dense_body,
        grid=grid,
        in_specs=[
            pl.BlockSpec((_ROWS_BLK, 256), lambda i: (i, 0)),
            full((256, 512)),
            full((1, 512)),
            full((512, 256)),
            full((1, 256)),
            full((256, 768)),
            full((1, 768)),
            full((1, 768)),
            full((256, 256)),
            full((1, 256)),
        ],
        out_specs=[
            pl.BlockSpec((_ROWS_BLK, 256), lambda i: (i, 0)),
            pl.BlockSpec((_ROWS_BLK, 256), lambda i: (i, 0)),
        ],
        out_shape=[
            jax.ShapeDtypeStruct((_N, 256), jnp.float32),
            jax.ShapeDtypeStruct((_N, 256), jnp.float32),
        ],
    )(s_cat, W2, b2, mix_W, mix_b, Wih, bih, bhh, ro_W, ro_b)


def kernel(x, edge_index, edge_attr, W0, b0, W1, b1, mix_W, mix_b,
           gru_Wih, gru_Whh, gru_bih, gru_bhh, ro_W, ro_b):
    del gru_Whh
    src = edge_index[0]
    dst = edge_index[1]
    ety = edge_attr

    cdst = dst + _N * ety
    csrc = src + _N * ety

    def _pack(g, sidx, goff1):
        gp = jnp.concatenate(
            [g, jnp.arange(_EPAD, dtype=jnp.int32) % _N])
        sp = jnp.concatenate(
            [sidx, 2 * _N + jnp.arange(_EPAD, dtype=jnp.int32) % (_NR - 2 * _N)])
        g2 = jnp.stack([gp, gp + goff1]).reshape(2, _NT * _NCHT, 1, _K)
        s2 = jnp.broadcast_to(
            sp.reshape(1, _NT * _NCHT, 1, _K), (2, _NT * _NCHT, 1, _K))
        return jnp.concatenate([g2, s2], axis=2).reshape(-1, 2, _K)

    pk1 = _pack(src, cdst, _N)
    pk2 = _pack(cdst, csrc, _NR)

    xs = jnp.concatenate([x[:, :64], x[:, 64:]], axis=0)

    s, _t = _sc_fused(xs, pk1, pk2)

    s_cat = jnp.concatenate(
        [s[0:_N], s[_NR:_NR + _N], s[_N:2 * _N], s[_NR + _N:_NR + 2 * _N]],
        axis=1)

    W2 = jnp.zeros((256, 512), jnp.float32)
    W2 = W2.at[:128, :256].set(W0).at[128:, 256:].set(W1)
    b2 = jnp.concatenate([b0, b1])[None, :]

    h_next, o = _dense_stage(
        s_cat, W2, b2, mix_W, mix_b[None, :], gru_Wih, gru_bih[None, :],
        gru_bhh[None, :], ro_W, ro_b[None, :])
    return (h_next, o[:, :3])


# speedup vs baseline: 1.0115x; 1.0115x over previous
import functools

import jax
import jax.numpy as jnp
from jax import lax
from jax.experimental import pallas as pl
from jax.experimental.pallas import tpu as pltpu
from jax.experimental.pallas import tpu_sc as plsc

_N = 10000
_E = 320000
_HID = 256
_ROWS_BLK = 2000

_NSC = 2
_NT = 16
_NR = 20480
_RT = _NR // _NT
_EP = _E // _NT
_K = 128
_NCHT = 158
_EPAD = _NT * _NCHT * _K - _E
_SB = 80


def _sc_fused_body(xs, pk1, pk2, s_out, t_out, acc, hist, pkA, pkB, rowsA,
                   rowsB, sb, histv, onesv, semA, semB):
    c = lax.axis_index("c")
    s = lax.axis_index("s")
    r0 = s * _RT
    cb = (c * _NT + s) * _NCHT
    z16 = jnp.zeros((16,), jnp.float32)
    one16 = jnp.ones((16,), jnp.float32)
    n_strips = _RT // _SB
    lane_splats = [jnp.full((16, 1), r, jnp.int32) for r in range(16)]
    _gd = lax.GatherDimensionNumbers(
        offset_dims=(), collapsed_slice_dims=(0,), start_index_map=(0,))

    def _zero_acc_hist():
        def _zstrip(st, carry):
            pltpu.sync_copy(rowsA.at[pl.ds(0, _SB)],
                            acc.at[pl.ds(r0 + st * _SB, _SB)])
            return carry
        lax.fori_loop(0, n_strips, _zstrip, 0)
        pltpu.sync_copy(histv, hist.at[pl.ds(r0, _RT)])

    def _edge_loop(table, pk):
        pltpu.sync_copy(pk.at[cb], pkA)
        pltpu.async_copy(table.at[pkA.at[0]], rowsA, semA)
        pltpu.sync_copy(pk.at[cb + 1], pkB)
        pltpu.async_copy(table.at[pkB.at[0]], rowsB, semB)

        def _pair(p, carry):
            a = 2 * p
            pltpu.make_async_copy(table.at[pl.ds(0, _K)], rowsA, semA).wait()
            pltpu.sync_copy(rowsA, acc.at[pkA.at[1]], add=True)
            pltpu.sync_copy(onesv, hist.at[pkA.at[1]], add=True)

            @pl.when(a + 2 < _NCHT)
            def _():
                pltpu.sync_copy(pk.at[cb + a + 2], pkA)
                pltpu.async_copy(table.at[pkA.at[0]], rowsA, semA)

            pltpu.make_async_copy(table.at[pl.ds(0, _K)], rowsB, semB).wait()
            pltpu.sync_copy(rowsB, acc.at[pkB.at[1]], add=True)
            pltpu.sync_copy(onesv, hist.at[pkB.at[1]], add=True)

            @pl.when(a + 3 < _NCHT)
            def _():
                pltpu.sync_copy(pk.at[cb + a + 3], pkB)
                pltpu.async_copy(table.at[pkB.at[0]], rowsB, semB)
            return carry
        lax.fori_loop(0, _NCHT // 2, _pair, 0)

    def _finalize(dst, rezero):
        if rezero:
            def _rz(i, carry):
                for j in range(4):
                    rowsA[i, pl.ds(j * 16, 16)] = z16
                return carry
            lax.fori_loop(0, _SB, _rz, 0)
        pltpu.sync_copy(hist.at[pl.ds(r0, _RT)], histv)

        def _inv(g, carry):
            hv = histv[pl.ds(g * 16, 16)]
            histv[pl.ds(g * 16, 16)] = jnp.where(hv > 0.0, 1.0 / hv, 0.0)
            return carry
        lax.fori_loop(0, _RT // 16, _inv, 0)

        def _fstrip(st, carry):
            pltpu.sync_copy(acc.at[pl.ds(r0 + st * _SB, _SB)], sb)

            def _grp(g, carry2):
                inv16 = histv[pl.ds(st * _SB + g * 16, 16)]
                for r in range(16):
                    splat = lax.gather(
                        inv16, lane_splats[r], _gd, slice_sizes=(1,),
                        mode=lax.GatherScatterMode.PROMISE_IN_BOUNDS)
                    row = g * 16 + r
                    for j in range(4):
                        sb[row, pl.ds(j * 16, 16)] = sb[row, pl.ds(j * 16, 16)] * splat
                return carry2
            lax.fori_loop(0, _SB // 16, _grp, 0)
            pltpu.sync_copy(sb, dst.at[pl.ds(c * _NR + r0 + st * _SB, _SB)])
            if rezero:
                pltpu.sync_copy(rowsA.at[pl.ds(0, _SB)],
                                acc.at[pl.ds(r0 + st * _SB, _SB)])
            return carry
        lax.fori_loop(0, n_strips, _fstrip, 0)
        if rezero:
            def _zh(g, carry):
                histv[pl.ds(g * 16, 16)] = z16
                return carry
            lax.fori_loop(0, _RT // 16, _zh, 0)
            pltpu.sync_copy(histv, hist.at[pl.ds(r0, _RT)])

    def _zrow(i, carry):
        for j in range(4):
            rowsA[i, pl.ds(j * 16, 16)] = z16
        return carry
    lax.fori_loop(0, _K, _zrow, 0)

    def _zhist(i, carry):
        histv[pl.ds(i * 16, 16)] = z16
        return carry
    lax.fori_loop(0, _RT // 16, _zhist, 0)
    for j in range(_K // 16):
        onesv[pl.ds(j * 16, 16)] = one16
    _zero_acc_hist()
    plsc.subcore_barrier()

    _edge_loop(xs, pk1)
    plsc.subcore_barrier()
    _finalize(t_out, rezero=True)
    plsc.subcore_barrier()

    _edge_loop(t_out, pk2)
    plsc.subcore_barrier()
    _finalize(s_out, rezero=False)


def _sc_fused(xs, pk1, pk2):
    mesh = plsc.VectorSubcoreMesh(core_axis_name="c", subcore_axis_name="s")
    f = pl.kernel(
        _sc_fused_body,
        mesh=mesh,
        out_type=[
            jax.ShapeDtypeStruct((_NSC * _NR, 64), jnp.float32),
            jax.ShapeDtypeStruct((_NSC * _NR, 64), jnp.float32),
        ],
        scratch_types=[
            pltpu.VMEM_SHARED((_NR, 64), jnp.float32),
            pltpu.VMEM_SHARED((_NR,), jnp.float32),
            pltpu.VMEM((2, _K), jnp.int32),
            pltpu.VMEM((2, _K), jnp.int32),
            pltpu.VMEM((_K, 64), jnp.float32),
            pltpu.VMEM((_K, 64), jnp.float32),
            pltpu.VMEM((_SB, 64), jnp.float32),
            pltpu.VMEM((_RT,), jnp.float32),
            pltpu.VMEM((_K,), jnp.float32),
            pltpu.SemaphoreType.DMA,
            pltpu.SemaphoreType.DMA,
        ],
        compiler_params=pltpu.CompilerParams(use_tc_tiling_on_sc=False),
    )
    return f(xs, pk1, pk2)


def _dense_body(s_ref, W2_ref, b2_ref, mixW_ref, mixb_ref,
                Wih_ref, bih_ref, bhh_ref, roW_ref, rob_ref, h_ref, o_ref):
    s = s_ref[...]
    u = jnp.dot(s, W2_ref[...], preferred_element_type=jnp.float32) + b2_ref[...]
    h = jnp.maximum(
        jnp.dot(u, mixW_ref[...], preferred_element_type=jnp.float32) + mixb_ref[...],
        0.0)
    gi = jnp.dot(h, Wih_ref[...], preferred_element_type=jnp.float32) + bih_ref[...]
    bhh = bhh_ref[...]
    r = jax.nn.sigmoid(gi[:, 0:_HID] + bhh[:, 0:_HID])
    z = jax.nn.sigmoid(gi[:, _HID:2 * _HID] + bhh[:, _HID:2 * _HID])
    n = jnp.tanh(gi[:, 2 * _HID:] + r * bhh[:, 2 * _HID:])
    hn = (1.0 - z) * n
    h_ref[...] = hn
    o_ref[...] = jnp.dot(hn, roW_ref[...], preferred_element_type=jnp.float32) + rob_ref[...]


def _dense_stage(s_cat, W2, b2, mix_W, mix_b, Wih, bih, bhh, ro_W, ro_b):
    grid = (_N // _ROWS_BLK,)
    full = lambda shape: pl.BlockSpec(shape, lambda i: (0, 0))
    return pl.pallas_call(
        _dense_body,
        grid=grid,
        in_specs=[
            pl.BlockSpec((_ROWS_BLK, 256), lambda i: (i, 0)),
            full((256, 512)),
            full((1, 512)),
            full((512, 256)),
            full((1, 256)),
            full((256, 768)),
            full((1, 768)),
            full((1, 768)),
            full((256, 256)),
            full((1, 256)),
        ],
        out_specs=[
            pl.BlockSpec((_ROWS_BLK, 256), lambda i: (i, 0)),
            pl.BlockSpec((_ROWS_BLK, 256), lambda i: (i, 0)),
        ],
        out_shape=[
            jax.ShapeDtypeStruct((_N, 256), jnp.float32),
            jax.ShapeDtypeStruct((_N, 256), jnp.float32),
        ],
    )(s_cat, W2, b2, mix_W, mix_b, Wih, bih, bhh, ro_W, ro_b)


def kernel(x, edge_index, edge_attr, W0, b0, W1, b1, mix_W, mix_b,
           gru_Wih, gru_Whh, gru_bih, gru_bhh, ro_W, ro_b):
    del gru_Whh
    src = edge_index[0]
    dst = edge_index[1]
    ety = edge_attr

    cdst = dst + _N * ety
    csrc = src + _N * ety

    def _pack(g, sidx, goff1):
        gp = jnp.concatenate(
            [g, jnp.arange(_EPAD, dtype=jnp.int32) % _N])
        sp = jnp.concatenate(
            [sidx, 2 * _N + jnp.arange(_EPAD, dtype=jnp.int32) % (_NR - 2 * _N)])
        g2 = jnp.stack([gp, gp + goff1]).reshape(2, _NT * _NCHT, 1, _K)
        s2 = jnp.broadcast_to(
            sp.reshape(1, _NT * _NCHT, 1, _K), (2, _NT * _NCHT, 1, _K))
        return jnp.concatenate([g2, s2], axis=2).reshape(-1, 2, _K)

    pk1 = _pack(src, cdst, _N)
    pk2 = _pack(cdst, csrc, _NR)

    xs = jnp.concatenate([x[:, :64], x[:, 64:]], axis=0)

    s, _t = _sc_fused(xs, pk1, pk2)

    s_cat = jnp.concatenate(
        [s[0:_N], s[_NR:_NR + _N], s[_N:2 * _N], s[_NR + _N:_NR + 2 * _N]],
        axis=1)

    W2 = jnp.zeros((256, 512), jnp.float32)
    W2 = W2.at[:128, :256].set(W0).at[128:, 256:].set(W1)
    b2 = jnp.concatenate([b0, b1])[None, :]

    h_next, o = _dense_stage(
        s_cat, W2, b2, mix_W, mix_b[None, :], gru_Wih, gru_bih[None, :],
        gru_bhh[None, :], ro_W, ro_b[None, :])
    return (h_next, o[:, :3])


# speedup vs baseline: 1.0178x; 1.0062x over previous
import functools

import jax
import jax.numpy as jnp
from jax import lax
from jax.experimental import pallas as pl
from jax.experimental.pallas import tpu as pltpu
from jax.experimental.pallas import tpu_sc as plsc

_N = 10000
_E = 320000
_HID = 256
_ROWS_BLK = 2000

_NSC = 2
_NT = 16
_NR = 20480
_RT = _NR // _NT
_EP = _E // _NT
_K = 128
_NCHT = 158
_EPAD = _NT * _NCHT * _K - _E
_SB = 128


def _sc_fused_body(xs, pk1, pk2, s_out, t_out, acc, hist, pkA, pkB, rowsA,
                   rowsB, sb, histv, onesv, semA, semB):
    c = lax.axis_index("c")
    s = lax.axis_index("s")
    r0 = s * _RT
    cb = (c * _NT + s) * _NCHT
    z16 = jnp.zeros((16,), jnp.float32)
    one16 = jnp.ones((16,), jnp.float32)
    n_strips = _RT // _SB
    lane_splats = [jnp.full((16, 1), r, jnp.int32) for r in range(16)]
    _gd = lax.GatherDimensionNumbers(
        offset_dims=(), collapsed_slice_dims=(0,), start_index_map=(0,))

    def _zero_acc_hist():
        def _zstrip(st, carry):
            pltpu.sync_copy(rowsA.at[pl.ds(0, _SB)],
                            acc.at[pl.ds(r0 + st * _SB, _SB)])
            return carry
        lax.fori_loop(0, n_strips, _zstrip, 0)
        pltpu.sync_copy(histv, hist.at[pl.ds(r0, _RT)])

    def _edge_loop(table, pk):
        pltpu.sync_copy(pk.at[cb], pkA)
        pltpu.async_copy(table.at[pkA.at[0]], rowsA, semA)
        pltpu.sync_copy(pk.at[cb + 1], pkB)
        pltpu.async_copy(table.at[pkB.at[0]], rowsB, semB)

        def _pair(p, carry):
            a = 2 * p
            pltpu.make_async_copy(table.at[pl.ds(0, _K)], rowsA, semA).wait()
            pltpu.sync_copy(rowsA, acc.at[pkA.at[1]], add=True)
            pltpu.sync_copy(onesv, hist.at[pkA.at[1]], add=True)

            @pl.when(a + 2 < _NCHT)
            def _():
                pltpu.sync_copy(pk.at[cb + a + 2], pkA)
                pltpu.async_copy(table.at[pkA.at[0]], rowsA, semA)

            pltpu.make_async_copy(table.at[pl.ds(0, _K)], rowsB, semB).wait()
            pltpu.sync_copy(rowsB, acc.at[pkB.at[1]], add=True)
            pltpu.sync_copy(onesv, hist.at[pkB.at[1]], add=True)

            @pl.when(a + 3 < _NCHT)
            def _():
                pltpu.sync_copy(pk.at[cb + a + 3], pkB)
                pltpu.async_copy(table.at[pkB.at[0]], rowsB, semB)
            return carry
        lax.fori_loop(0, _NCHT // 2, _pair, 0)

    def _finalize(dst, rezero):
        if rezero:
            def _rz(i, carry):
                for j in range(4):
                    rowsA[i, pl.ds(j * 16, 16)] = z16
                return carry
            lax.fori_loop(0, _SB, _rz, 0)
        pltpu.sync_copy(hist.at[pl.ds(r0, _RT)], histv)

        def _inv(g, carry):
            hv = histv[pl.ds(g * 16, 16)]
            histv[pl.ds(g * 16, 16)] = jnp.where(hv > 0.0, 1.0 / hv, 0.0)
            return carry
        lax.fori_loop(0, _RT // 16, _inv, 0)

        def _fstrip(st, carry):
            pltpu.sync_copy(acc.at[pl.ds(r0 + st * _SB, _SB)], sb)

            def _grp(g, carry2):
                inv16 = histv[pl.ds(st * _SB + g * 16, 16)]
                for r in range(16):
                    splat = lax.gather(
                        inv16, lane_splats[r], _gd, slice_sizes=(1,),
                        mode=lax.GatherScatterMode.PROMISE_IN_BOUNDS)
                    row = g * 16 + r
                    for j in range(4):
                        sb[row, pl.ds(j * 16, 16)] = sb[row, pl.ds(j * 16, 16)] * splat
                return carry2
            lax.fori_loop(0, _SB // 16, _grp, 0)
            pltpu.sync_copy(sb, dst.at[pl.ds(c * _NR + r0 + st * _SB, _SB)])
            if rezero:
                pltpu.sync_copy(rowsA.at[pl.ds(0, _SB)],
                                acc.at[pl.ds(r0 + st * _SB, _SB)])
            return carry
        lax.fori_loop(0, n_strips, _fstrip, 0)
        if rezero:
            def _zh(g, carry):
                histv[pl.ds(g * 16, 16)] = z16
                return carry
            lax.fori_loop(0, _RT // 16, _zh, 0)
            pltpu.sync_copy(histv, hist.at[pl.ds(r0, _RT)])

    def _zrow(i, carry):
        for j in range(4):
            rowsA[i, pl.ds(j * 16, 16)] = z16
        return carry
    lax.fori_loop(0, _K, _zrow, 0)

    def _zhist(i, carry):
        histv[pl.ds(i * 16, 16)] = z16
        return carry
    lax.fori_loop(0, _RT // 16, _zhist, 0)
    for j in range(_K // 16):
        onesv[pl.ds(j * 16, 16)] = one16
    _zero_acc_hist()
    plsc.subcore_barrier()

    _edge_loop(xs, pk1)
    plsc.subcore_barrier()
    _finalize(t_out, rezero=True)
    plsc.subcore_barrier()

    _edge_loop(t_out, pk2)
    plsc.subcore_barrier()
    _finalize(s_out, rezero=False)


def _sc_fused(xs, pk1, pk2):
    mesh = plsc.VectorSubcoreMesh(core_axis_name="c", subcore_axis_name="s")
    f = pl.kernel(
        _sc_fused_body,
        mesh=mesh,
        out_type=[
            jax.ShapeDtypeStruct((_NSC * _NR, 64), jnp.float32),
            jax.ShapeDtypeStruct((_NSC * _NR, 64), jnp.float32),
        ],
        scratch_types=[
            pltpu.VMEM_SHARED((_NR, 64), jnp.float32),
            pltpu.VMEM_SHARED((_NR,), jnp.float32),
            pltpu.VMEM((2, _K), jnp.int32),
            pltpu.VMEM((2, _K), jnp.int32),
            pltpu.VMEM((_K, 64), jnp.float32),
            pltpu.VMEM((_K, 64), jnp.float32),
            pltpu.VMEM((_SB, 64), jnp.float32),
            pltpu.VMEM((_RT,), jnp.float32),
            pltpu.VMEM((_K,), jnp.float32),
            pltpu.SemaphoreType.DMA,
            pltpu.SemaphoreType.DMA,
        ],
        compiler_params=pltpu.CompilerParams(use_tc_tiling_on_sc=False),
    )
    return f(xs, pk1, pk2)


def _dense_body(s_ref, W2_ref, b2_ref, mixW_ref, mixb_ref,
                Wih_ref, bih_ref, bhh_ref, roW_ref, rob_ref, h_ref, o_ref):
    s = s_ref[...]
    u = jnp.dot(s, W2_ref[...], preferred_element_type=jnp.float32) + b2_ref[...]
    h = jnp.maximum(
        jnp.dot(u, mixW_ref[...], preferred_element_type=jnp.float32) + mixb_ref[...],
        0.0)
    gi = jnp.dot(h, Wih_ref[...], preferred_element_type=jnp.float32) + bih_ref[...]
    bhh = bhh_ref[...]
    r = jax.nn.sigmoid(gi[:, 0:_HID] + bhh[:, 0:_HID])
    z = jax.nn.sigmoid(gi[:, _HID:2 * _HID] + bhh[:, _HID:2 * _HID])
    n = jnp.tanh(gi[:, 2 * _HID:] + r * bhh[:, 2 * _HID:])
    hn = (1.0 - z) * n
    h_ref[...] = hn
    o_ref[...] = jnp.dot(hn, roW_ref[...], preferred_element_type=jnp.float32) + rob_ref[...]


def _dense_stage(s_cat, W2, b2, mix_W, mix_b, Wih, bih, bhh, ro_W, ro_b):
    grid = (_N // _ROWS_BLK,)
    full = lambda shape: pl.BlockSpec(shape, lambda i: (0, 0))
    return pl.pallas_call(
        _dense_body,
        grid=grid,
        in_specs=[
            pl.BlockSpec((_ROWS_BLK, 256), lambda i: (i, 0)),
            full((256, 512)),
            full((1, 512)),
            full((512, 256)),
            full((1, 256)),
            full((256, 768)),
            full((1, 768)),
            full((1, 768)),
            full((256, 256)),
            full((1, 256)),
        ],
        out_specs=[
            pl.BlockSpec((_ROWS_BLK, 256), lambda i: (i, 0)),
            pl.BlockSpec((_ROWS_BLK, 256), lambda i: (i, 0)),
        ],
        out_shape=[
            jax.ShapeDtypeStruct((_N, 256), jnp.float32),
            jax.ShapeDtypeStruct((_N, 256), jnp.float32),
        ],
    )(s_cat, W2, b2, mix_W, mix_b, Wih, bih, bhh, ro_W, ro_b)


def kernel(x, edge_index, edge_attr, W0, b0, W1, b1, mix_W, mix_b,
           gru_Wih, gru_Whh, gru_bih, gru_bhh, ro_W, ro_b):
    del gru_Whh
    src = edge_index[0]
    dst = edge_index[1]
    ety = edge_attr

    cdst = dst + _N * ety
    csrc = src + _N * ety

    def _pack(g, sidx, goff1):
        gp = jnp.concatenate(
            [g, jnp.arange(_EPAD, dtype=jnp.int32) % _N])
        sp = jnp.concatenate(
            [sidx, 2 * _N + jnp.arange(_EPAD, dtype=jnp.int32) % (_NR - 2 * _N)])
        g2 = jnp.stack([gp, gp + goff1]).reshape(2, _NT * _NCHT, 1, _K)
        s2 = jnp.broadcast_to(
            sp.reshape(1, _NT * _NCHT, 1, _K), (2, _NT * _NCHT, 1, _K))
        return jnp.concatenate([g2, s2], axis=2).reshape(-1, 2, _K)

    pk1 = _pack(src, cdst, _N)
    pk2 = _pack(cdst, csrc, _NR)

    xs = jnp.concatenate([x[:, :64], x[:, 64:]], axis=0)

    s, _t = _sc_fused(xs, pk1, pk2)

    s_cat = jnp.concatenate(
        [s[0:_N], s[_NR:_NR + _N], s[_N:2 * _N], s[_NR + _N:_NR + 2 * _N]],
        axis=1)

    W2 = jnp.zeros((256, 512), jnp.float32)
    W2 = W2.at[:128, :256].set(W0).at[128:, 256:].set(W1)
    b2 = jnp.concatenate([b0, b1])[None, :]

    h_next, o = _dense_stage(
        s_cat, W2, b2, mix_W, mix_b[None, :], gru_Wih, gru_bih[None, :],
        gru_bhh[None, :], ro_W, ro_b[None, :])
    return (h_next, o[:, :3])


# speedup vs baseline: 1.0180x; 1.0002x over previous
import jax
import jax.numpy as jnp
from jax import lax
from jax.experimental import pallas as pl
from jax.experimental.pallas import tpu as pltpu
from jax.experimental.pallas import tpu_sc as plsc

_N = 10000
_E = 320000
_HID = 256
_ROWS_BLK = 2000

_NSC = 2
_NT = 16
_NR = 20480
_RT = _NR // _NT
_K = 128
_NCHT = 158
_EPAD = _NT * _NCHT * _K - _E
_SB = 128


def _sc_fused_body(xs, pk1, pk2, s_out, t_out, acc, hist, pkA, pkB, rowsA,
                   rowsB, sb, histv, onesv, semA, semB):
    c = lax.axis_index("c")
    s = lax.axis_index("s")
    r0 = s * _RT
    cb = (c * _NT + s) * _NCHT
    z16 = jnp.zeros((16,), jnp.float32)
    one16 = jnp.ones((16,), jnp.float32)
    n_strips = _RT // _SB
    lane_splats = [jnp.full((16, 1), r, jnp.int32) for r in range(16)]
    _gd = lax.GatherDimensionNumbers(
        offset_dims=(), collapsed_slice_dims=(0,), start_index_map=(0,))

    def _zero_acc_hist():
        def _zstrip(st, carry):
            pltpu.sync_copy(rowsA.at[pl.ds(0, _SB)],
                            acc.at[pl.ds(r0 + st * _SB, _SB)])
            return carry
        lax.fori_loop(0, n_strips, _zstrip, 0)
        pltpu.sync_copy(histv, hist.at[pl.ds(r0, _RT)])

    def _edge_loop(table, pk):
        pltpu.sync_copy(pk.at[cb], pkA)
        pltpu.async_copy(table.at[pkA.at[0]], rowsA, semA)
        pltpu.sync_copy(pk.at[cb + 1], pkB)
        pltpu.async_copy(table.at[pkB.at[0]], rowsB, semB)

        def _pair(p, carry):
            a = 2 * p
            pltpu.make_async_copy(table.at[pl.ds(0, _K)], rowsA, semA).wait()
            pltpu.sync_copy(rowsA, acc.at[pkA.at[1]], add=True)
            pltpu.sync_copy(onesv, hist.at[pkA.at[1]], add=True)

            @pl.when(a + 2 < _NCHT)
            def _():
                pltpu.sync_copy(pk.at[cb + a + 2], pkA)
                pltpu.async_copy(table.at[pkA.at[0]], rowsA, semA)

            pltpu.make_async_copy(table.at[pl.ds(0, _K)], rowsB, semB).wait()
            pltpu.sync_copy(rowsB, acc.at[pkB.at[1]], add=True)
            pltpu.sync_copy(onesv, hist.at[pkB.at[1]], add=True)

            @pl.when(a + 3 < _NCHT)
            def _():
                pltpu.sync_copy(pk.at[cb + a + 3], pkB)
                pltpu.async_copy(table.at[pkB.at[0]], rowsB, semB)
            return carry
        lax.fori_loop(0, _NCHT // 2, _pair, 0)

    def _finalize(dst, rezero):
        if rezero:
            def _rz(i, carry):
                for j in range(4):
                    rowsA[i, pl.ds(j * 16, 16)] = z16
                return carry
            lax.fori_loop(0, _SB, _rz, 0)
        pltpu.sync_copy(hist.at[pl.ds(r0, _RT)], histv)

        def _inv(g, carry):
            hv = histv[pl.ds(g * 16, 16)]
            histv[pl.ds(g * 16, 16)] = jnp.where(hv > 0.0, 1.0 / hv, 0.0)
            return carry
        lax.fori_loop(0, _RT // 16, _inv, 0)

        def _fstrip(st, carry):
            pltpu.sync_copy(acc.at[pl.ds(r0 + st * _SB, _SB)], sb)

            def _grp(g, carry2):
                inv16 = histv[pl.ds(st * _SB + g * 16, 16)]
                for r in range(16):
                    splat = lax.gather(
                        inv16, lane_splats[r], _gd, slice_sizes=(1,),
                        mode=lax.GatherScatterMode.PROMISE_IN_BOUNDS)
                    row = g * 16 + r
                    for j in range(4):
                        sb[row, pl.ds(j * 16, 16)] = sb[row, pl.ds(j * 16, 16)] * splat
                return carry2
            lax.fori_loop(0, _SB // 16, _grp, 0)
            pltpu.sync_copy(sb, dst.at[pl.ds(c * _NR + r0 + st * _SB, _SB)])
            if rezero:
                pltpu.sync_copy(rowsA.at[pl.ds(0, _SB)],
                                acc.at[pl.ds(r0 + st * _SB, _SB)])
            return carry
        lax.fori_loop(0, n_strips, _fstrip, 0)
        if rezero:
            def _zh(g, carry):
                histv[pl.ds(g * 16, 16)] = z16
                return carry
            lax.fori_loop(0, _RT // 16, _zh, 0)
            pltpu.sync_copy(histv, hist.at[pl.ds(r0, _RT)])

    def _zrow(i, carry):
        for j in range(4):
            rowsA[i, pl.ds(j * 16, 16)] = z16
        return carry
    lax.fori_loop(0, _K, _zrow, 0)

    def _zhist(i, carry):
        histv[pl.ds(i * 16, 16)] = z16
        return carry
    lax.fori_loop(0, _RT // 16, _zhist, 0)
    for j in range(_K // 16):
        onesv[pl.ds(j * 16, 16)] = one16
    _zero_acc_hist()
    plsc.subcore_barrier()

    _edge_loop(xs, pk1)
    plsc.subcore_barrier()
    _finalize(t_out, rezero=True)
    plsc.subcore_barrier()

    _edge_loop(t_out, pk2)
    plsc.subcore_barrier()
    _finalize(s_out, rezero=False)


def _sc_fused(xs, pk1, pk2):
    mesh = plsc.VectorSubcoreMesh(core_axis_name="c", subcore_axis_name="s")
    f = pl.kernel(
        _sc_fused_body,
        mesh=mesh,
        out_type=[
            jax.ShapeDtypeStruct((_NSC * _NR, 64), jnp.float32),
            jax.ShapeDtypeStruct((_NSC * _NR, 64), jnp.float32),
        ],
        scratch_types=[
            pltpu.VMEM_SHARED((_NR, 64), jnp.float32),
            pltpu.VMEM_SHARED((_NR,), jnp.float32),
            pltpu.VMEM((2, _K), jnp.int32),
            pltpu.VMEM((2, _K), jnp.int32),
            pltpu.VMEM((_K, 64), jnp.float32),
            pltpu.VMEM((_K, 64), jnp.float32),
            pltpu.VMEM((_SB, 64), jnp.float32),
            pltpu.VMEM((_RT,), jnp.float32),
            pltpu.VMEM((_K,), jnp.float32),
            pltpu.SemaphoreType.DMA,
            pltpu.SemaphoreType.DMA,
        ],
        compiler_params=pltpu.CompilerParams(use_tc_tiling_on_sc=False),
    )
    return f(xs, pk1, pk2)


def _dense_body(s_ref, W2_ref, b2_ref, mixW_ref, mixb_ref,
                Wih_ref, bih_ref, bhh_ref, roW_ref, rob_ref, h_ref, o_ref):
    s = s_ref[...]
    u = jnp.dot(s, W2_ref[...], preferred_element_type=jnp.float32) + b2_ref[...]
    h = jnp.maximum(
        jnp.dot(u, mixW_ref[...], preferred_element_type=jnp.float32) + mixb_ref[...],
        0.0)
    gi = jnp.dot(h, Wih_ref[...], preferred_element_type=jnp.float32) + bih_ref[...]
    bhh = bhh_ref[...]
    r = jax.nn.sigmoid(gi[:, 0:_HID] + bhh[:, 0:_HID])
    z = jax.nn.sigmoid(gi[:, _HID:2 * _HID] + bhh[:, _HID:2 * _HID])
    n = jnp.tanh(gi[:, 2 * _HID:] + r * bhh[:, 2 * _HID:])
    hn = (1.0 - z) * n
    h_ref[...] = hn
    o_ref[...] = jnp.dot(hn, roW_ref[...], preferred_element_type=jnp.float32) + rob_ref[...]


def _dense_stage(s_cat, W2, b2, mix_W, mix_b, Wih, bih, bhh, ro_W, ro_b):
    grid = (_N // _ROWS_BLK,)
    full = lambda shape: pl.BlockSpec(shape, lambda i: (0, 0))
    return pl.pallas_call(
        _dense_body,
        grid=grid,
        in_specs=[
            pl.BlockSpec((_ROWS_BLK, 256), lambda i: (i, 0)),
            full((256, 512)),
            full((1, 512)),
            full((512, 256)),
            full((1, 256)),
            full((256, 768)),
            full((1, 768)),
            full((1, 768)),
            full((256, 256)),
            full((1, 256)),
        ],
        out_specs=[
            pl.BlockSpec((_ROWS_BLK, 256), lambda i: (i, 0)),
            pl.BlockSpec((_ROWS_BLK, 256), lambda i: (i, 0)),
        ],
        out_shape=[
            jax.ShapeDtypeStruct((_N, 256), jnp.float32),
            jax.ShapeDtypeStruct((_N, 256), jnp.float32),
        ],
    )(s_cat, W2, b2, mix_W, mix_b, Wih, bih, bhh, ro_W, ro_b)


def kernel(x, edge_index, edge_attr, W0, b0, W1, b1, mix_W, mix_b,
           gru_Wih, gru_Whh, gru_bih, gru_bhh, ro_W, ro_b):
    del gru_Whh
    src = edge_index[0]
    dst = edge_index[1]
    ety = edge_attr

    cdst = dst + _N * ety
    csrc = src + _N * ety

    def _pack(g, sidx, goff1):
        gp = jnp.concatenate(
            [g, jnp.arange(_EPAD, dtype=jnp.int32) % _N])
        sp = jnp.concatenate(
            [sidx, 2 * _N + jnp.arange(_EPAD, dtype=jnp.int32) % (_NR - 2 * _N)])
        g2 = jnp.stack([gp, gp + goff1]).reshape(2, _NT * _NCHT, 1, _K)
        s2 = jnp.broadcast_to(
            sp.reshape(1, _NT * _NCHT, 1, _K), (2, _NT * _NCHT, 1, _K))
        return jnp.concatenate([g2, s2], axis=2).reshape(-1, 2, _K)

    pk1 = _pack(src, cdst, _N)
    pk2 = _pack(cdst, csrc, _NR)

    xs = jnp.concatenate([x[:, :64], x[:, 64:]], axis=0)

    s, _t = _sc_fused(xs, pk1, pk2)

    s_cat = jnp.concatenate(
        [s[0:_N], s[_NR:_NR + _N], s[_N:2 * _N], s[_NR + _N:_NR + 2 * _N]],
        axis=1)

    W2 = jnp.zeros((256, 512), jnp.float32)
    W2 = W2.at[:128, :256].set(W0).at[128:, 256:].set(W1)
    b2 = jnp.concatenate([b0, b1])[None, :]

    h_next, o = _dense_stage(
        s_cat, W2, b2, mix_W, mix_b[None, :], gru_Wih, gru_bih[None, :],
        gru_bhh[None, :], ro_W, ro_b[None, :])
    return (h_next, o[:, :3])
